# Initial kernel scaffold; baseline (speedup 1.0000x reference)
import jax
import jax.numpy as jnp
from jax.experimental import pallas as pl


def kernel(qlen, klen, table):
    raise NotImplementedError("write your pallas kernel here")


# speedup vs baseline: 42.5244x; 42.5244x over previous
import functools

import jax
import jax.numpy as jnp
from jax import lax
from jax.experimental import pallas as pl
from jax.experimental.pallas import tpu as pltpu
from jax.experimental.pallas import tpu_sc as plsc

H = 16
Q = 2048
K = 2048
NB = 32
L = 16
VLEN = 4112
WROW = 4096
THRESH = (12, 16, 23, 32, 46, 64, 91)


def _sc_body(tab_hbm, out_hbm, tab_v, v_v, w_v, sem):
    c = lax.axis_index("c")
    s = lax.axis_index("s")
    wid = s * 2 + c
    head = wid // 2
    i0 = (wid % 2) * (Q // 2)

    pltpu.sync_copy(tab_hbm, tab_v)

    def vbody(g, carry):
        t = g * L + lax.iota(jnp.int32, L)
        d = t - (Q - 1)
        n = -d
        one = jnp.full((L,), 1, jnp.int32)
        zero = jnp.full((L,), 0, jnp.int32)
        side = jnp.where(n < 0, one * (NB // 2), zero)
        m = jnp.abs(n)
        big = jnp.full((L,), 8, jnp.int32)
        for th in THRESH:
            big = big + jnp.where(m >= th, one, zero)
        bk = side + jnp.where(m < 8, m, big)
        vals = plsc.load_gather(tab_v, [bk * H + head])
        v_v[pl.ds(g * L, L)] = vals
        return carry

    lax.fori_loop(0, VLEN // L, vbody, 0)

    for r in range(8):
        off = 7 - r

        def wbody(g, carry, r=r, off=off):
            w_v[pl.ds(r * WROW + g * L, L)] = v_v[pl.ds(g * L + off, L)]
            return carry

        lax.fori_loop(0, WROW // L, wbody, 0)

    q0 = i0 // 8

    def drain8():
        for _ in range(8):
            pltpu.make_async_copy(
                w_v.at[pl.ds(0, K)], out_hbm.at[pl.ds(0, K)], sem
            ).wait()

    def mbody(q, carry):
        base = 2040 - 8 * (q0 + q)
        for r in range(8):
            row = head * Q + i0 + 8 * q + r
            pltpu.async_copy(
                w_v.at[pl.ds(r * WROW + base, K)],
                out_hbm.at[pl.ds(row * K, K)],
                sem,
            )

        @pl.when(q > 0)
        def _():
            drain8()

        return carry

    lax.fori_loop(0, (Q // 2) // 8, mbody, 0)
    drain8()


def kernel(qlen, klen, table):
    mesh = plsc.VectorSubcoreMesh(core_axis_name="c", subcore_axis_name="s")
    run = pl.kernel(
        _sc_body,
        out_type=jax.ShapeDtypeStruct((H * Q * K,), jnp.float32),
        mesh=mesh,
        scratch_types=[
            pltpu.VMEM((NB * H,), jnp.float32),
            pltpu.VMEM((VLEN,), jnp.float32),
            pltpu.VMEM((8 * WROW,), jnp.float32),
            pltpu.SemaphoreType.DMA,
        ],
        compiler_params=pltpu.CompilerParams(needs_layout_passes=False),
    )
    flat = run(table.reshape(NB * H))
    return flat.reshape(1, H, Q, K)
